# baseline (speedup 1.0000x reference)
import functools
import math

import jax
import jax.numpy as jnp
from jax import lax
from jax.experimental import pallas as pl
from jax.experimental.pallas import tpu as pltpu
from jax.experimental.pallas import tpu_sc as plsc

D_MODEL = 64
NUM_WORKERS = 32
CHUNK = 128


def _sc_gather(table, idx3):
    nw, n_chunks, chunk = idx3.shape
    b_per_w = n_chunks * chunk
    B = nw * b_per_w

    mesh = plsc.VectorSubcoreMesh(core_axis_name="c", subcore_axis_name="s")

    @functools.partial(
        pl.kernel,
        out_type=jax.ShapeDtypeStruct((B, D_MODEL), jnp.float32),
        mesh=mesh,
        scratch_types=[
            pltpu.VMEM((n_chunks, chunk), jnp.int32),
            pltpu.VMEM((chunk, D_MODEL), jnp.float32),
            pltpu.SemaphoreType.DMA,
        ],
        compiler_params=pltpu.CompilerParams(use_tc_tiling_on_sc=False),
    )
    def gather_kernel(table_hbm, idx_hbm, out_hbm, idx_v, rows_v, sem):
        wid = lax.axis_index("s") * 2 + lax.axis_index("c")
        base = wid * b_per_w
        pltpu.sync_copy(idx_hbm.at[wid], idx_v)

        def body(j, carry):
            pltpu.async_copy(table_hbm.at[idx_v.at[j]], rows_v, sem).wait()
            pltpu.sync_copy(rows_v, out_hbm.at[pl.ds(base + j * chunk, chunk)])
            return carry

        lax.fori_loop(0, n_chunks, body, 0)

    return gather_kernel(table, idx3)


def _tc_adapter(rows, w_t8, b8):
    B = rows.shape[0]
    blk = 4096

    def body(x_ref, w_ref, b_ref, o_ref):
        o_ref[...] = (
            jnp.dot(x_ref[...], w_ref[...], preferred_element_type=jnp.float32)
            + b_ref[...]
        )

    return pl.pallas_call(
        body,
        grid=(B // blk,),
        in_specs=[
            pl.BlockSpec((blk, D_MODEL), lambda i: (i, 0)),
            pl.BlockSpec((D_MODEL, D_MODEL), lambda i: (0, 0)),
            pl.BlockSpec((1, D_MODEL), lambda i: (0, 0)),
        ],
        out_specs=pl.BlockSpec((blk, D_MODEL), lambda i: (i, 0)),
        out_shape=jax.ShapeDtypeStruct((B, D_MODEL), jnp.float32),
    )(rows, w_t8, b8)


def kernel(x, table, W, b, lang_id):
    bsz, seqlen = x.shape
    B = bsz * seqlen
    n_chunks = B // (NUM_WORKERS * CHUNK)
    idx3 = x.reshape(NUM_WORKERS, n_chunks, CHUNK).astype(jnp.int32)

    rows = _sc_gather(table, idx3)

    scale = math.sqrt(float(D_MODEL))
    w_t8 = W.T * scale
    b8 = (b * scale).reshape(1, D_MODEL)
    y = _tc_adapter(rows, w_t8, b8)
    return y.reshape(bsz, seqlen, D_MODEL)


# speedup vs baseline: 1.0853x; 1.0853x over previous
import functools
import math

import jax
import jax.numpy as jnp
from jax import lax
from jax.experimental import pallas as pl
from jax.experimental.pallas import tpu as pltpu
from jax.experimental.pallas import tpu_sc as plsc

D_MODEL = 64
NUM_WORKERS = 32
CHUNK = 128


def _sc_gather(table, idx2, n_chunks):
    chunk = idx2.shape[1]
    b_per_w = n_chunks * chunk
    B = NUM_WORKERS * b_per_w

    mesh = plsc.VectorSubcoreMesh(core_axis_name="c", subcore_axis_name="s")

    @functools.partial(
        pl.kernel,
        out_type=jax.ShapeDtypeStruct((B, D_MODEL), jnp.float32),
        mesh=mesh,
        scratch_types=[
            pltpu.VMEM((n_chunks, chunk), jnp.int32),
            pltpu.VMEM((chunk, D_MODEL), jnp.float32),
            pltpu.SemaphoreType.DMA,
        ],
        compiler_params=pltpu.CompilerParams(use_tc_tiling_on_sc=False),
    )
    def gather_kernel(table_hbm, idx_hbm, out_hbm, idx_v, rows_v, sem):
        wid = lax.axis_index("s") * 2 + lax.axis_index("c")
        base = wid * b_per_w
        pltpu.sync_copy(idx_hbm.at[pl.ds(wid * n_chunks, n_chunks)], idx_v)

        def body(j, carry):
            pltpu.async_copy(table_hbm.at[idx_v.at[j]], rows_v, sem).wait()
            pltpu.sync_copy(rows_v, out_hbm.at[pl.ds(base + j * chunk, chunk)])
            return carry

        lax.fori_loop(0, n_chunks, body, 0)

    return gather_kernel(table, idx2)


def _tc_adapter(pairs, w_big, b_big, bsz, seqlen):
    rows_blk = 32
    pair_blk = rows_blk * seqlen // 2

    def body(x_ref, w_ref, b_ref, o_ref):
        o_ref[...] = (
            jnp.dot(x_ref[...], w_ref[...], preferred_element_type=jnp.float32)
            + b_ref[...]
        )

    n_pairs = bsz * seqlen // 2
    y2 = pl.pallas_call(
        body,
        grid=(bsz // rows_blk,),
        in_specs=[
            pl.BlockSpec((pair_blk, 2 * D_MODEL), lambda i: (i, 0)),
            pl.BlockSpec((2 * D_MODEL, 2 * D_MODEL), lambda i: (0, 0)),
            pl.BlockSpec((1, 2 * D_MODEL), lambda i: (0, 0)),
        ],
        out_specs=pl.BlockSpec((pair_blk, 2 * D_MODEL), lambda i: (i, 0)),
        out_shape=jax.ShapeDtypeStruct((n_pairs, 2 * D_MODEL), jnp.float32),
    )(pairs, w_big, b_big)
    return y2.reshape(bsz, seqlen, D_MODEL)


def kernel(x, table, W, b, lang_id):
    bsz, seqlen = x.shape
    B = bsz * seqlen
    n_chunks = B // (NUM_WORKERS * CHUNK)
    idx2 = x.reshape(NUM_WORKERS * n_chunks, CHUNK).astype(jnp.int32)

    rows = _sc_gather(table, idx2, n_chunks)

    scale = math.sqrt(float(D_MODEL))
    w_t8 = W.T * scale
    zeros = jnp.zeros_like(w_t8)
    w_big = jnp.block([[w_t8, zeros], [zeros, w_t8]])
    b8 = b * scale
    b_big = jnp.concatenate([b8, b8]).reshape(1, 2 * D_MODEL)

    pairs = rows.reshape(B // 2, 2 * D_MODEL)
    return _tc_adapter(pairs, w_big, b_big, bsz, seqlen)


# speedup vs baseline: 1.6334x; 1.5051x over previous
import functools
import math

import jax
import jax.numpy as jnp
from jax import lax
from jax.experimental import pallas as pl
from jax.experimental.pallas import tpu as pltpu
from jax.experimental.pallas import tpu_sc as plsc

D_MODEL = 64
NUM_WORKERS = 32
CHUNK = 128


SPLIT = 524288
BLKW = 8192


def _tc_transform(table_t, w_t8, b8):
    V = table_t.shape[1]
    nblk = SPLIT // BLKW
    last_blk = (V + BLKW - 1) // BLKW - 1

    def body(lo_ref, hi_ref, w_ref, b_ref, o_ref):
        dn = (((0,), (0,)), ((), ()))
        lo = lax.dot_general(lo_ref[...], w_ref[...], dn,
                             preferred_element_type=jnp.float32) + b_ref[...]
        hi = lax.dot_general(hi_ref[...], w_ref[...], dn,
                             preferred_element_type=jnp.float32) + b_ref[...]
        o_ref[...] = jnp.concatenate([lo, hi], axis=1)

    return pl.pallas_call(
        body,
        grid=(nblk,),
        in_specs=[
            pl.BlockSpec((D_MODEL, BLKW), lambda i: (0, i)),
            pl.BlockSpec(
                (D_MODEL, BLKW),
                lambda i, n=nblk, lb=last_blk: (0, jnp.minimum(i + n, lb)),
            ),
            pl.BlockSpec((D_MODEL, D_MODEL), lambda i: (0, 0)),
            pl.BlockSpec((1, D_MODEL), lambda i: (0, 0)),
        ],
        out_specs=pl.BlockSpec((BLKW, 2 * D_MODEL), lambda i: (i, 0)),
        out_shape=jax.ShapeDtypeStruct((SPLIT, 2 * D_MODEL), jnp.float32),
    )(table_t, table_t, w_t8, b8)


def _sc_gather(table, idx2, n_chunks):
    chunk = idx2.shape[1]
    b_per_w = n_chunks * chunk
    B = NUM_WORKERS * b_per_w

    mesh = plsc.VectorSubcoreMesh(core_axis_name="c", subcore_axis_name="s")

    @functools.partial(
        pl.kernel,
        out_type=jax.ShapeDtypeStruct((B, D_MODEL), jnp.float32),
        mesh=mesh,
        scratch_types=[
            pltpu.VMEM((n_chunks, chunk), jnp.int32),
            pltpu.VMEM((chunk, D_MODEL), jnp.float32),
            pltpu.SemaphoreType.DMA,
        ],
        compiler_params=pltpu.CompilerParams(use_tc_tiling_on_sc=False),
    )
    def gather_kernel(table_hbm, idx_hbm, out_hbm, idx_v, rows_v, sem):
        wid = lax.axis_index("s") * 2 + lax.axis_index("c")
        base = wid * b_per_w
        pltpu.sync_copy(idx_hbm.at[pl.ds(wid * n_chunks, n_chunks)], idx_v)

        def body(j, carry):
            pltpu.async_copy(table_hbm.at[idx_v.at[j]], rows_v, sem).wait()
            pltpu.sync_copy(rows_v, out_hbm.at[pl.ds(base + j * chunk, chunk)])
            return carry

        lax.fori_loop(0, n_chunks, body, 0)

    return gather_kernel(table, idx2)


def kernel(x, table, W, b, lang_id):
    bsz, seqlen = x.shape
    B = bsz * seqlen
    n_chunks = B // (NUM_WORKERS * CHUNK)

    scale = math.sqrt(float(D_MODEL))
    w_t8 = W.T * scale
    b8 = (b * scale).reshape(1, D_MODEL)

    t2_pairs = _tc_transform(table.T, w_t8, b8)
    t2 = t2_pairs.reshape(2 * SPLIT, D_MODEL)

    xi = x.reshape(-1).astype(jnp.int32)
    g = jnp.where(xi < SPLIT, 2 * xi, 2 * (xi - SPLIT) + 1)
    idx2 = g.reshape(NUM_WORKERS * n_chunks, CHUNK)

    y = _sc_gather(t2, idx2, n_chunks)
    return y.reshape(bsz, seqlen, D_MODEL)


# speedup vs baseline: 1.7130x; 1.0487x over previous
import functools
import math

import jax
import jax.numpy as jnp
from jax import lax
from jax.experimental import pallas as pl
from jax.experimental.pallas import tpu as pltpu
from jax.experimental.pallas import tpu_sc as plsc

D_MODEL = 64
NUM_WORKERS = 32
CHUNK = 128


SPLIT = 524288
BLKW = 8192


def _tc_transform(table_t, w_t8, b8):
    V = table_t.shape[1]
    nblk = SPLIT // BLKW
    last_blk = (V + BLKW - 1) // BLKW - 1

    def body(lo_ref, hi_ref, w_ref, b_ref, o_ref):
        dn = (((0,), (0,)), ((), ()))
        lo = lax.dot_general(lo_ref[...], w_ref[...], dn,
                             preferred_element_type=jnp.float32) + b_ref[...]
        hi = lax.dot_general(hi_ref[...], w_ref[...], dn,
                             preferred_element_type=jnp.float32) + b_ref[...]
        o_ref[...] = jnp.concatenate([lo, hi], axis=1)

    return pl.pallas_call(
        body,
        grid=(nblk,),
        in_specs=[
            pl.BlockSpec((D_MODEL, BLKW), lambda i: (0, i)),
            pl.BlockSpec(
                (D_MODEL, BLKW),
                lambda i, n=nblk, lb=last_blk: (0, jnp.minimum(i + n, lb)),
            ),
            pl.BlockSpec((D_MODEL, D_MODEL), lambda i: (0, 0)),
            pl.BlockSpec((1, D_MODEL), lambda i: (0, 0)),
        ],
        out_specs=pl.BlockSpec((BLKW, 2 * D_MODEL), lambda i: (i, 0)),
        out_shape=jax.ShapeDtypeStruct((SPLIT, 2 * D_MODEL), jnp.float32),
    )(table_t, table_t, w_t8, b8)


def _sc_gather(table, idx2, bsz, seqlen):
    chunk = idx2.shape[1]
    nb = bsz // NUM_WORKERS

    mesh = plsc.VectorSubcoreMesh(core_axis_name="c", subcore_axis_name="s")

    @functools.partial(
        pl.kernel,
        out_type=jax.ShapeDtypeStruct((bsz, seqlen, D_MODEL), jnp.float32),
        mesh=mesh,
        scratch_types=[
            pltpu.VMEM((2 * nb, chunk), jnp.int32),
            pltpu.VMEM((seqlen, D_MODEL), jnp.float32),
            pltpu.SemaphoreType.DMA,
        ],
        compiler_params=pltpu.CompilerParams(use_tc_tiling_on_sc=False),
    )
    def gather_kernel(table_hbm, idx_hbm, out_hbm, idx_v, stage_v, sem):
        wid = lax.axis_index("s") * 2 + lax.axis_index("c")
        pltpu.sync_copy(idx_hbm.at[pl.ds(wid * 2 * nb, 2 * nb)], idx_v)

        def body(lb, carry):
            cp0 = pltpu.async_copy(
                table_hbm.at[idx_v.at[2 * lb]],
                stage_v.at[pl.ds(0, chunk)], sem)
            cp1 = pltpu.async_copy(
                table_hbm.at[idx_v.at[2 * lb + 1]],
                stage_v.at[pl.ds(chunk, chunk)], sem)
            cp0.wait()
            cp1.wait()
            pltpu.sync_copy(stage_v, out_hbm.at[wid * nb + lb])
            return carry

        lax.fori_loop(0, nb, body, 0)

    return gather_kernel(table, idx2)


def kernel(x, table, W, b, lang_id):
    bsz, seqlen = x.shape

    scale = math.sqrt(float(D_MODEL))
    w_t8 = W.T * scale
    b8 = (b * scale).reshape(1, D_MODEL)

    t2_pairs = _tc_transform(table.T, w_t8, b8)
    t2 = t2_pairs.reshape(2 * SPLIT, D_MODEL)

    xi = x.reshape(-1).astype(jnp.int32)
    g = jnp.where(xi < SPLIT, 2 * xi, 2 * (xi - SPLIT) + 1)
    idx2 = g.reshape(bsz * 2, seqlen // 2)

    return _sc_gather(t2, idx2, bsz, seqlen)


# speedup vs baseline: 1.7369x; 1.0140x over previous
import functools
import math

import jax
import jax.numpy as jnp
from jax import lax
from jax.experimental import pallas as pl
from jax.experimental.pallas import tpu as pltpu
from jax.experimental.pallas import tpu_sc as plsc

D_MODEL = 64
NUM_WORKERS = 32
CHUNK = 128


SPLIT = 524288
BLKW = 8192


def _tc_transform(table_t, w_t8, b8):
    V = table_t.shape[1]
    nblk = SPLIT // BLKW
    last_blk = (V + BLKW - 1) // BLKW - 1

    def body(lo_ref, hi_ref, w_ref, b_ref, o_ref):
        dn = (((0,), (0,)), ((), ()))
        lo = lax.dot_general(lo_ref[...], w_ref[...], dn,
                             preferred_element_type=jnp.float32) + b_ref[...]
        hi = lax.dot_general(hi_ref[...], w_ref[...], dn,
                             preferred_element_type=jnp.float32) + b_ref[...]
        o_ref[...] = jnp.concatenate([lo, hi], axis=1)

    return pl.pallas_call(
        body,
        grid=(nblk,),
        in_specs=[
            pl.BlockSpec((D_MODEL, BLKW), lambda i: (0, i)),
            pl.BlockSpec(
                (D_MODEL, BLKW),
                lambda i, n=nblk, lb=last_blk: (0, jnp.minimum(i + n, lb)),
            ),
            pl.BlockSpec((D_MODEL, D_MODEL), lambda i: (0, 0)),
            pl.BlockSpec((1, D_MODEL), lambda i: (0, 0)),
        ],
        out_specs=pl.BlockSpec((BLKW, 2 * D_MODEL), lambda i: (i, 0)),
        out_shape=jax.ShapeDtypeStruct((SPLIT, 2 * D_MODEL), jnp.float32),
    )(table_t, table_t, w_t8, b8)


def _sc_gather(table, idx2, bsz, seqlen):
    chunk = idx2.shape[1]
    nb = bsz // NUM_WORKERS

    mesh = plsc.VectorSubcoreMesh(core_axis_name="c", subcore_axis_name="s")

    @functools.partial(
        pl.kernel,
        out_type=jax.ShapeDtypeStruct((bsz * seqlen, D_MODEL), jnp.float32),
        mesh=mesh,
        scratch_types=[
            pltpu.VMEM((2 * nb, chunk), jnp.int32),
            pltpu.VMEM((seqlen, D_MODEL), jnp.float32),
            pltpu.SemaphoreType.DMA,
        ],
        compiler_params=pltpu.CompilerParams(use_tc_tiling_on_sc=False),
    )
    def gather_kernel(table_hbm, idx_hbm, out_hbm, idx_v, stage_v, sem):
        wid = lax.axis_index("s") * 2 + lax.axis_index("c")
        pltpu.sync_copy(idx_hbm.at[pl.ds(wid * 2 * nb, 2 * nb)], idx_v)

        def body(lb, carry):
            cp0 = pltpu.async_copy(
                table_hbm.at[idx_v.at[2 * lb]],
                stage_v.at[pl.ds(0, chunk)], sem)
            cp1 = pltpu.async_copy(
                table_hbm.at[idx_v.at[2 * lb + 1]],
                stage_v.at[pl.ds(chunk, chunk)], sem)
            cp0.wait()
            cp1.wait()
            pltpu.sync_copy(
                stage_v, out_hbm.at[pl.ds((wid * nb + lb) * seqlen, seqlen)])
            return carry

        lax.fori_loop(0, nb, body, 0)

    return gather_kernel(table, idx2)


def _tc_to_output_layout(y3, bsz, seqlen):
    BL = 128
    AW = 10
    cols = AW * 2 * D_MODEL
    y6 = y3.reshape(bsz, seqlen * D_MODEL)

    def body(x_ref, o_ref):
        xt = x_ref[...].T
        o_ref[...] = xt.reshape(2 * AW, D_MODEL, BL)

    zt = pl.pallas_call(
        body,
        grid=(bsz // BL, (seqlen // 2) // AW),
        in_specs=[pl.BlockSpec((BL, cols), lambda j, k: (j, k))],
        out_specs=pl.BlockSpec((2 * AW, D_MODEL, BL), lambda j, k: (k, 0, j)),
        out_shape=jax.ShapeDtypeStruct((seqlen, D_MODEL, bsz), jnp.float32),
    )(y6)
    return zt.transpose(2, 0, 1)


def kernel(x, table, W, b, lang_id):
    bsz, seqlen = x.shape

    scale = math.sqrt(float(D_MODEL))
    w_t8 = W.T * scale
    b8 = (b * scale).reshape(1, D_MODEL)

    t2_pairs = _tc_transform(table.T, w_t8, b8)
    t2 = t2_pairs.reshape(2 * SPLIT, D_MODEL)

    xi = x.reshape(-1).astype(jnp.int32)
    g = jnp.where(xi < SPLIT, 2 * xi, 2 * (xi - SPLIT) + 1)
    idx2 = g.reshape(bsz * 2, seqlen // 2)

    y3 = _sc_gather(t2, idx2, bsz, seqlen)
    return _tc_to_output_layout(y3, bsz, seqlen)
